# baseline (speedup 1.0000x reference)
import functools

import jax
import jax.numpy as jnp
from jax import lax
from jax.experimental import pallas as pl
from jax.experimental.pallas import tpu as pltpu
from jax.experimental.pallas import tpu_sc as plsc

B = 16384
F = 26
V = 100000
D = 16
H = 256
FD = F * D

NC = 2
NS = 16
L = 16
NW = NC * NS
BPW = B // NW
IPW = BPW * F
CHUNK = 128
NCH = IPW // CHUNK

_mesh = plsc.VectorSubcoreMesh(
    core_axis_name="c", subcore_axis_name="s", num_cores=NC, num_subcores=NS
)


@functools.partial(
    pl.kernel,
    out_type=(
        jax.ShapeDtypeStruct((B * F * D // 128, 128), jnp.float32),
        jax.ShapeDtypeStruct((B * F,), jnp.float32),
    ),
    mesh=_mesh,
    scratch_types=(
        pltpu.VMEM((NCH, CHUNK), jnp.int32),
        pltpu.VMEM((NCH, CHUNK), jnp.int32),
        pltpu.VMEM((NCH, CHUNK), jnp.int32),
        pltpu.VMEM((CHUNK, 128), jnp.float32),
        pltpu.VMEM((CHUNK, 128), jnp.float32),
        pltpu.VMEM((CHUNK * D // 128, 128), jnp.float32),
        pltpu.VMEM((CHUNK,), jnp.float32),
        pltpu.SemaphoreType.DMA,
        pltpu.SemaphoreType.DMA,
    ),
    compiler_params=pltpu.CompilerParams(
        use_tc_tiling_on_sc=True, needs_layout_passes=False
    ),
)
def _sc_gather(idx_hbm, emb_hbm, widew_hbm, embout_hbm, wideout_hbm,
               idx_v, gdx_v, wdx_v, rows_v, wrows_v, sel_v, wsel_v,
               sem_g, sem_w):
    wid = lax.axis_index("s") * NC + lax.axis_index("c")

    pltpu.sync_copy(idx_hbm.at[pl.ds(wid * NCH, NCH)], idx_v)

    groups_per_row = CHUNK // L

    def _fix(i, carry):
        r = i // groups_per_row
        c = (i % groups_per_row) * L
        pos = i * L + lax.iota(jnp.int32, L)
        f = lax.rem(pos, F)
        flat = idx_v[r, pl.ds(c, L)] + f * V
        idx_v[r, pl.ds(c, L)] = flat
        gdx_v[r, pl.ds(c, L)] = lax.shift_right_logical(flat, 3)
        wdx_v[r, pl.ds(c, L)] = lax.shift_right_logical(flat, 7)
        return carry

    lax.fori_loop(0, NCH * groups_per_row, _fix, 0)

    lanes = lax.iota(jnp.int32, L)

    def _chunk(s, carry):
        wd = pltpu.async_copy(widew_hbm.at[wdx_v.at[s]], wrows_v, sem_w)
        ed = pltpu.async_copy(emb_hbm.at[gdx_v.at[s]], rows_v, sem_g)
        ed.wait()
        def _sel(g, c2):
            mv = lax.bitwise_and(idx_v[s, pl.ds(g * L, L)], 7)
            for u in range(L):
                j = g * L + u
                sel_v[g * 2 + u // 8, pl.ds((u % 8) * D, D)] = rows_v[
                    j, pl.ds(mv[u] * D, D)
                ]
            return c2

        lax.fori_loop(0, CHUNK // L, _sel, 0)
        out_off = (wid * NCH + s) * (CHUNK * D // 128)
        pltpu.sync_copy(sel_v, embout_hbm.at[pl.ds(out_off, CHUNK * D // 128)])
        wd.wait()
        for g in range(groups_per_row):
            m = lax.bitwise_and(idx_v[s, pl.ds(g * L, L)], 127)
            wsel_v[pl.ds(g * L, L)] = plsc.load_gather(
                wrows_v, [g * L + lanes, m]
            )
        pltpu.sync_copy(wsel_v, wideout_hbm.at[pl.ds((wid * NCH + s) * CHUNK, CHUNK)])
        return carry

    lax.fori_loop(0, NCH, _chunk, 0)


_BM = 2048


def _mlp_body(x_ref, wide_ref, w1_ref, b1_ref, w2_ref, b2_ref, wf_ref, bf_ref,
              o_ref):
    x = x_ref[...]
    h = jnp.maximum(jnp.dot(x, w1_ref[...]) + b1_ref[...], 0.0)
    h = jnp.maximum(jnp.dot(h, w2_ref[...]) + b2_ref[...], 0.0)
    d = jnp.dot(h, wf_ref[...]) + bf_ref[...]
    wsum = jnp.sum(wide_ref[...], axis=1, keepdims=True)
    o_ref[...] = jax.nn.sigmoid(0.5 * wsum + 0.5 * d)


_mlp = pl.pallas_call(
    _mlp_body,
    grid=(B // _BM,),
    in_specs=[
        pl.BlockSpec((_BM, FD), lambda i: (i, 0)),
        pl.BlockSpec((_BM, F), lambda i: (i, 0)),
        pl.BlockSpec((FD, H), lambda i: (0, 0)),
        pl.BlockSpec((1, H), lambda i: (0, 0)),
        pl.BlockSpec((H, H), lambda i: (0, 0)),
        pl.BlockSpec((1, H), lambda i: (0, 0)),
        pl.BlockSpec((H, 1), lambda i: (0, 0)),
        pl.BlockSpec((1, 1), lambda i: (0, 0)),
    ],
    out_specs=pl.BlockSpec((_BM, 1), lambda i: (i, 0)),
    out_shape=jax.ShapeDtypeStruct((B, 1), jnp.float32),
)


def kernel(inputs, embed_tables, W1, b1, W2, b2, Wf, bf, wide_w):
    idx = inputs.astype(jnp.int32).reshape(NW * NCH, CHUNK)
    emb_rows = embed_tables.reshape(F * V * D // 128, 128)
    npad = (-(F * V)) % 128
    wide_rows = jnp.pad(wide_w.reshape(-1), (0, npad)).reshape(-1, 128)
    embed_out, wide_out = _sc_gather(idx, emb_rows, wide_rows)
    x = embed_out.reshape(B, FD)
    return _mlp(
        x,
        wide_out.reshape(B, F),
        W1,
        b1.reshape(1, H),
        W2,
        b2.reshape(1, H),
        Wf,
        bf.reshape(1, 1),
    )


# speedup vs baseline: 1.4506x; 1.4506x over previous
import functools

import jax
import jax.numpy as jnp
from jax import lax
from jax.experimental import pallas as pl
from jax.experimental.pallas import tpu as pltpu
from jax.experimental.pallas import tpu_sc as plsc

B = 16384
F = 26
V = 100000
D = 16
H = 256
FD = F * D

NC = 2
NS = 16
L = 16
NW = NC * NS
BPW = B // NW
IPW = BPW * F
CHUNK = 128
NCH = IPW // CHUNK

_mesh = plsc.VectorSubcoreMesh(
    core_axis_name="c", subcore_axis_name="s", num_cores=NC, num_subcores=NS
)


@functools.partial(
    pl.kernel,
    out_type=(
        jax.ShapeDtypeStruct((B * F * D // 128, 128), jnp.float32),
        jax.ShapeDtypeStruct((B * F,), jnp.float32),
    ),
    mesh=_mesh,
    scratch_types=(
        pltpu.VMEM((NCH, CHUNK), jnp.int32),
        pltpu.VMEM((NCH, CHUNK), jnp.int32),
        pltpu.VMEM((NCH, CHUNK), jnp.int32),
        pltpu.VMEM((CHUNK, 128), jnp.float32),
        pltpu.VMEM((CHUNK, 128), jnp.float32),
        pltpu.VMEM((CHUNK * D // 128, 128), jnp.float32),
        pltpu.VMEM((CHUNK,), jnp.float32),
        pltpu.SemaphoreType.DMA,
        pltpu.SemaphoreType.DMA,
    ),
    compiler_params=pltpu.CompilerParams(
        use_tc_tiling_on_sc=True, needs_layout_passes=False
    ),
)
def _sc_gather(idx_hbm, emb_hbm, widew_hbm, embout_hbm, wideout_hbm,
               idx_v, gdx_v, wdx_v, rows_v, wrows_v, sel_v, wsel_v,
               sem_g, sem_w):
    wid = lax.axis_index("s") * NC + lax.axis_index("c")

    pltpu.sync_copy(idx_hbm.at[pl.ds(wid * NCH, NCH)], idx_v)

    groups_per_row = CHUNK // L

    def _fix(i, carry):
        r = i // groups_per_row
        c = (i % groups_per_row) * L
        pos = i * L + lax.iota(jnp.int32, L)
        f = lax.rem(pos, F)
        flat = idx_v[r, pl.ds(c, L)] + f * V
        idx_v[r, pl.ds(c, L)] = flat
        gdx_v[r, pl.ds(c, L)] = lax.shift_right_logical(flat, 3)
        wdx_v[r, pl.ds(c, L)] = lax.shift_right_logical(flat, 7)
        return carry

    lax.fori_loop(0, NCH * groups_per_row, _fix, 0)

    lanes = lax.iota(jnp.int32, L)

    def _chunk(s, carry):
        wd = pltpu.async_copy(widew_hbm.at[wdx_v.at[s]], wrows_v, sem_w)
        ed = pltpu.async_copy(emb_hbm.at[gdx_v.at[s]], rows_v, sem_g)
        ed.wait()
        def _sel(g, c2):
            mv = lax.bitwise_and(idx_v[s, pl.ds(g * L, L)], 7)
            for u in range(L):
                j = g * L + u
                sel_v[g * 2 + u // 8, pl.ds((u % 8) * D, D)] = rows_v[
                    j, pl.ds(mv[u] * D, D)
                ]
            return c2

        lax.fori_loop(0, CHUNK // L, _sel, 0)
        out_off = (wid * NCH + s) * (CHUNK * D // 128)
        pltpu.sync_copy(sel_v, embout_hbm.at[pl.ds(out_off, CHUNK * D // 128)])
        wd.wait()
        for g in range(groups_per_row):
            m = lax.bitwise_and(idx_v[s, pl.ds(g * L, L)], 127)
            wsel_v[pl.ds(g * L, L)] = plsc.load_gather(
                wrows_v, [g * L + lanes, m]
            )
        pltpu.sync_copy(wsel_v, wideout_hbm.at[pl.ds((wid * NCH + s) * CHUNK, CHUNK)])
        return carry

    lax.fori_loop(0, NCH, _chunk, 0)


NP_FULL = F * (V // 128)
PPW = -(-NP_FULL // NW)
VT_LAST = V // 128
RPT = V // 8


@functools.partial(
    pl.kernel,
    out_type=jax.ShapeDtypeStruct((F, V * D // 128, 128), jnp.float32),
    mesh=_mesh,
    scratch_types=(
        pltpu.VMEM((2, 2, 8, 128), jnp.float32),
        pltpu.VMEM((2, 16, 128), jnp.float32),
        pltpu.VMEM((2, 2, 8, 32), jnp.float32),
        pltpu.VMEM((2, 4, 128), jnp.float32),
        pltpu.SemaphoreType.DMA,
        pltpu.SemaphoreType.DMA,
        pltpu.SemaphoreType.DMA,
        pltpu.SemaphoreType.DMA,
    ),
    compiler_params=pltpu.CompilerParams(
        use_tc_tiling_on_sc=True, needs_layout_passes=False
    ),
)
def _sc_transpose(nat_hbm, out_hbm, tin_v, tout_v, uin_v, uout_v,
                  sg0, sg1, ss0, ss1):
    wid = lax.axis_index("s") * NC + lax.axis_index("c")
    lanes = lax.iota(jnp.int32, L)
    rowpat = lax.shift_right_logical(lanes, 3)
    colpat = lax.bitwise_and(lanes, 7) * D
    sgs = (sg0, sg1)
    sss = (ss0, ss1)
    lo = wid * PPW
    hi = jnp.minimum(lo + PPW, NP_FULL)

    def _src_slice(p, t):
        f = p // VT_LAST
        vt = lax.rem(p, VT_LAST)
        return nat_hbm.at[
            pl.ds(pl.multiple_of(f * D + t * 8, 8), 8),
            pl.ds(pl.multiple_of(vt * 128, 128), 128),
        ]

    def _issue_load(p, slot):
        for t in range(2):
            pltpu.async_copy(_src_slice(p, t), tin_v.at[slot, t], sgs[slot])

    def _wait_load(p, slot):
        for t in range(2):
            pltpu.make_async_copy(
                _src_slice(p, t), tin_v.at[slot, t], sgs[slot]
            ).wait()

    def _store_slice(p, slot):
        f = p // VT_LAST
        vt = lax.rem(p, VT_LAST)
        return out_hbm.at[f, pl.ds(pl.multiple_of(vt * 16, 8), 16)]

    for b in range(2):
        @pl.when(lo + b < hi)
        def _():
            _issue_load(lo + b, b)

    def _body(i2, carry):
        for b in range(2):
            i = i2 * 2 + b
            p = lo + i

            @pl.when(p < hi)
            def _():
                _wait_load(p, b)

                @pl.when(i >= 2)
                def _():
                    pltpu.make_async_copy(
                        tout_v.at[b], _store_slice(p - 2, b), sss[b]
                    ).wait()

                for d in range(D):
                    for vb in range(8):
                        vec = tin_v[b, d // 8, d % 8, pl.ds(vb * L, L)]
                        plsc.store_scatter(
                            tout_v.at[b],
                            [vb * 2 + rowpat, colpat + d],
                            vec,
                        )
                pltpu.async_copy(tout_v.at[b], _store_slice(p, b), sss[b])

                @pl.when(p + 2 < hi)
                def _():
                    _issue_load(p + 2, b)

        return carry

    lax.fori_loop(0, (PPW + 1) // 2, _body, 0)

    for b in range(2):
        pb = jnp.where(lax.rem(hi - 1 - lo, 2) == b, hi - 1, hi - 2)

        @pl.when(pb >= lo)
        def _():
            pltpu.make_async_copy(
                tout_v.at[b], _store_slice(pb, b), sss[b]
            ).wait()

    @pl.when(wid < F)
    def _():
        f = wid
        for t in range(2):
            pltpu.async_copy(
                nat_hbm.at[
                    pl.ds(pl.multiple_of(f * D + t * 8, 8), 8),
                    pl.ds(VT_LAST * 128, 32),
                ],
                uin_v.at[0, t], sg0,
            )
        for t in range(2):
            pltpu.make_async_copy(
                nat_hbm.at[
                    pl.ds(pl.multiple_of(f * D + t * 8, 8), 8),
                    pl.ds(VT_LAST * 128, 32),
                ],
                uin_v.at[0, t], sg0,
            ).wait()
        for d in range(D):
            for vb in range(2):
                vec = uin_v[0, d // 8, d % 8, pl.ds(vb * L, L)]
                plsc.store_scatter(
                    uout_v.at[0],
                    [vb * 2 + rowpat, colpat + d],
                    vec,
                )
        pltpu.sync_copy(
            uout_v.at[0], out_hbm.at[f, pl.ds(VT_LAST * 16, 4)]
        )


_BM = 2048


def _mlp_body(x_ref, wide_ref, w1_ref, b1_ref, w2_ref, b2_ref, wf_ref, bf_ref,
              o_ref):
    x = x_ref[...]
    h = jnp.maximum(jnp.dot(x, w1_ref[...]) + b1_ref[...], 0.0)
    h = jnp.maximum(jnp.dot(h, w2_ref[...]) + b2_ref[...], 0.0)
    d = jnp.dot(h, wf_ref[...]) + bf_ref[...]
    wsum = jnp.sum(wide_ref[...], axis=1, keepdims=True)
    o_ref[...] = jax.nn.sigmoid(0.5 * wsum + 0.5 * d)


_mlp = pl.pallas_call(
    _mlp_body,
    grid=(B // _BM,),
    in_specs=[
        pl.BlockSpec((_BM, FD), lambda i: (i, 0)),
        pl.BlockSpec((_BM, F), lambda i: (i, 0)),
        pl.BlockSpec((FD, H), lambda i: (0, 0)),
        pl.BlockSpec((1, H), lambda i: (0, 0)),
        pl.BlockSpec((H, H), lambda i: (0, 0)),
        pl.BlockSpec((1, H), lambda i: (0, 0)),
        pl.BlockSpec((H, 1), lambda i: (0, 0)),
        pl.BlockSpec((1, 1), lambda i: (0, 0)),
    ],
    out_specs=pl.BlockSpec((_BM, 1), lambda i: (i, 0)),
    out_shape=jax.ShapeDtypeStruct((B, 1), jnp.float32),
)


def kernel(inputs, embed_tables, W1, b1, W2, b2, Wf, bf, wide_w):
    idx = inputs.astype(jnp.int32).reshape(NW * NCH, CHUNK)
    nat2d = jnp.swapaxes(embed_tables, 1, 2).reshape(F * D, V)
    emb_rows = _sc_transpose(nat2d).reshape(F * V * D // 128, 128)
    npad = (-(F * V)) % 128
    wide_rows = jnp.pad(wide_w.reshape(-1), (0, npad)).reshape(-1, 128)
    embed_out, wide_out = _sc_gather(idx, emb_rows, wide_rows)
    x = embed_out.reshape(B, FD)
    return _mlp(
        x,
        wide_out.reshape(B, F),
        W1,
        b1.reshape(1, H),
        W2,
        b2.reshape(1, H),
        Wf,
        bf.reshape(1, 1),
    )


# speedup vs baseline: 1.7783x; 1.2259x over previous
import functools

import jax
import jax.numpy as jnp
from jax import lax
from jax.experimental import pallas as pl
from jax.experimental.pallas import tpu as pltpu
from jax.experimental.pallas import tpu_sc as plsc

B = 16384
F = 26
V = 100000
D = 16
H = 256
FD = F * D

NC = 2
NS = 16
L = 16
NW = NC * NS
BPW = B // NW
IPW = BPW * F
CHUNK = 128
NCH = IPW // CHUNK

_mesh = plsc.VectorSubcoreMesh(
    core_axis_name="c", subcore_axis_name="s", num_cores=NC, num_subcores=NS
)


@functools.partial(
    pl.kernel,
    out_type=jax.ShapeDtypeStruct((B * F * D // 128, 128), jnp.float32),
    mesh=_mesh,
    scratch_types=(
        pltpu.VMEM((NCH, CHUNK), jnp.int32),
        pltpu.VMEM((NCH, CHUNK), jnp.int32),
        pltpu.VMEM((2, CHUNK, 128), jnp.float32),
        pltpu.VMEM((2, CHUNK * D // 128, 128), jnp.float32),
        pltpu.SemaphoreType.DMA,
        pltpu.SemaphoreType.DMA,
        pltpu.SemaphoreType.DMA,
        pltpu.SemaphoreType.DMA,
    ),
    compiler_params=pltpu.CompilerParams(
        use_tc_tiling_on_sc=True, needs_layout_passes=False
    ),
)
def _sc_gather(idx_hbm, emb_hbm, embout_hbm,
               idx_v, gdx_v, rows_v, sel_v, sg0, sg1, ss0, ss1):
    wid = lax.axis_index("s") * NC + lax.axis_index("c")
    sgs = (sg0, sg1)
    sss = (ss0, ss1)

    pltpu.sync_copy(idx_hbm.at[pl.ds(wid * NCH, NCH)], idx_v)

    groups_per_row = CHUNK // L

    def _fix(i, carry):
        r = i // groups_per_row
        c = (i % groups_per_row) * L
        pos = i * L + lax.iota(jnp.int32, L)
        f = lax.rem(pos, F)
        flat = idx_v[r, pl.ds(c, L)] + f * V
        idx_v[r, pl.ds(c, L)] = flat
        gdx_v[r, pl.ds(c, L)] = lax.shift_right_logical(flat, 3)
        return carry

    lax.fori_loop(0, NCH * groups_per_row, _fix, 0)

    OPC = CHUNK * D // 128

    def _gather(s, b):
        pltpu.async_copy(emb_hbm.at[gdx_v.at[s]], rows_v.at[b], sgs[b])

    def _wait_gather(s, b):
        pltpu.make_async_copy(emb_hbm.at[gdx_v.at[s]], rows_v.at[b], sgs[b]).wait()

    def _store_slice(s):
        return embout_hbm.at[pl.ds((wid * NCH + s) * OPC, OPC)]

    for b in range(2):
        _gather(b, b)

    def _chunk(s2, carry):
        for b in range(2):
            s = s2 * 2 + b

            @pl.when(s < NCH)
            def _():
                _wait_gather(s, b)

                @pl.when(s >= 2)
                def _():
                    pltpu.make_async_copy(
                        sel_v.at[b], _store_slice(s - 2), sss[b]
                    ).wait()

                def _sel(g, c2):
                    mv = lax.bitwise_and(idx_v[s, pl.ds(g * L, L)], 7)
                    for u in range(L):
                        j = g * L + u
                        sel_v[b, g * 2 + u // 8, pl.ds((u % 8) * D, D)] = rows_v[
                            b, j, pl.ds(mv[u] * D, D)
                        ]
                    return c2

                lax.fori_loop(0, CHUNK // L, _sel, 0)
                pltpu.async_copy(sel_v.at[b], _store_slice(s), sss[b])

                @pl.when(s + 2 < NCH)
                def _():
                    _gather(s + 2, b)

        return carry

    lax.fori_loop(0, NCH // 2, _chunk, 0)
    for b in range(2):
        pltpu.make_async_copy(
            sel_v.at[b], _store_slice(NCH - 2 + b), sss[b]
        ).wait()


@functools.partial(
    pl.kernel,
    out_type=jax.ShapeDtypeStruct((B * F,), jnp.float32),
    mesh=_mesh,
    scratch_types=(
        pltpu.VMEM((NCH, CHUNK), jnp.int32),
        pltpu.VMEM((NCH, CHUNK), jnp.int32),
        pltpu.VMEM((2, CHUNK, D), jnp.float32),
        pltpu.VMEM((2, CHUNK), jnp.float32),
        pltpu.SemaphoreType.DMA,
        pltpu.SemaphoreType.DMA,
        pltpu.SemaphoreType.DMA,
        pltpu.SemaphoreType.DMA,
    ),
    compiler_params=pltpu.CompilerParams(
        use_tc_tiling_on_sc=False, needs_layout_passes=False
    ),
)
def _sc_wide(idx_hbm, widew_hbm, wideout_hbm,
             idx_v, wdx_v, wrows_v, wsel_v, sg0, sg1, ss0, ss1):
    wid = lax.axis_index("s") * NC + lax.axis_index("c")
    sgs = (sg0, sg1)
    sss = (ss0, ss1)
    lanes = lax.iota(jnp.int32, L)
    pltpu.sync_copy(idx_hbm.at[pl.ds(wid * NCH, NCH)], idx_v)
    groups_per_row = CHUNK // L

    def _fix(i, carry):
        r = i // groups_per_row
        c = (i % groups_per_row) * L
        pos = i * L + lax.iota(jnp.int32, L)
        f = lax.rem(pos, F)
        flat = idx_v[r, pl.ds(c, L)] + f * V
        idx_v[r, pl.ds(c, L)] = flat
        wdx_v[r, pl.ds(c, L)] = lax.shift_right_logical(flat, 4)
        return carry

    lax.fori_loop(0, NCH * groups_per_row, _fix, 0)

    def _gather(s, b):
        pltpu.async_copy(widew_hbm.at[wdx_v.at[s]], wrows_v.at[b], sgs[b])

    def _wait_gather(s, b):
        pltpu.make_async_copy(
            widew_hbm.at[wdx_v.at[s]], wrows_v.at[b], sgs[b]
        ).wait()

    def _store_slice(s):
        return wideout_hbm.at[pl.ds((wid * NCH + s) * CHUNK, CHUNK)]

    for b in range(2):
        _gather(b, b)

    def _chunk(s2, carry):
        for b in range(2):
            s = s2 * 2 + b

            @pl.when(s < NCH)
            def _():
                _wait_gather(s, b)

                @pl.when(s >= 2)
                def _():
                    pltpu.make_async_copy(
                        wsel_v.at[b], _store_slice(s - 2), sss[b]
                    ).wait()

                for g in range(groups_per_row):
                    m = lax.bitwise_and(idx_v[s, pl.ds(g * L, L)], 15)
                    wsel_v[b, pl.ds(g * L, L)] = plsc.load_gather(
                        wrows_v.at[b], [g * L + lanes, m]
                    )
                pltpu.async_copy(wsel_v.at[b], _store_slice(s), sss[b])

                @pl.when(s + 2 < NCH)
                def _():
                    _gather(s + 2, b)

        return carry

    lax.fori_loop(0, NCH // 2, _chunk, 0)
    for b in range(2):
        pltpu.make_async_copy(
            wsel_v.at[b], _store_slice(NCH - 2 + b), sss[b]
        ).wait()


NP_FULL = F * (V // 128)
PPW = -(-NP_FULL // NW)
VT_LAST = V // 128
RPT = V // 8


@functools.partial(
    pl.kernel,
    out_type=jax.ShapeDtypeStruct((F, V * D // 128, 128), jnp.float32),
    mesh=_mesh,
    scratch_types=(
        pltpu.VMEM((2, 2, 8, 128), jnp.float32),
        pltpu.VMEM((2, 16, 128), jnp.float32),
        pltpu.VMEM((2, 2, 8, 32), jnp.float32),
        pltpu.VMEM((2, 4, 128), jnp.float32),
        pltpu.SemaphoreType.DMA,
        pltpu.SemaphoreType.DMA,
        pltpu.SemaphoreType.DMA,
        pltpu.SemaphoreType.DMA,
    ),
    compiler_params=pltpu.CompilerParams(
        use_tc_tiling_on_sc=True, needs_layout_passes=False
    ),
)
def _sc_transpose(nat_hbm, out_hbm, tin_v, tout_v, uin_v, uout_v,
                  sg0, sg1, ss0, ss1):
    wid = lax.axis_index("s") * NC + lax.axis_index("c")
    lanes = lax.iota(jnp.int32, L)
    rowpat = lax.shift_right_logical(lanes, 3)
    colpat = lax.bitwise_and(lanes, 7) * D
    sgs = (sg0, sg1)
    sss = (ss0, ss1)
    lo = wid * PPW
    hi = jnp.minimum(lo + PPW, NP_FULL)

    def _src_slice(p, t):
        f = p // VT_LAST
        vt = lax.rem(p, VT_LAST)
        return nat_hbm.at[
            pl.ds(pl.multiple_of(f * D + t * 8, 8), 8),
            pl.ds(pl.multiple_of(vt * 128, 128), 128),
        ]

    def _issue_load(p, slot):
        for t in range(2):
            pltpu.async_copy(_src_slice(p, t), tin_v.at[slot, t], sgs[slot])

    def _wait_load(p, slot):
        for t in range(2):
            pltpu.make_async_copy(
                _src_slice(p, t), tin_v.at[slot, t], sgs[slot]
            ).wait()

    def _store_slice(p, slot):
        f = p // VT_LAST
        vt = lax.rem(p, VT_LAST)
        return out_hbm.at[f, pl.ds(pl.multiple_of(vt * 16, 8), 16)]

    for b in range(2):
        @pl.when(lo + b < hi)
        def _():
            _issue_load(lo + b, b)

    def _body(i2, carry):
        for b in range(2):
            i = i2 * 2 + b
            p = lo + i

            @pl.when(p < hi)
            def _():
                _wait_load(p, b)

                @pl.when(i >= 2)
                def _():
                    pltpu.make_async_copy(
                        tout_v.at[b], _store_slice(p - 2, b), sss[b]
                    ).wait()

                for d in range(D):
                    for vb in range(8):
                        vec = tin_v[b, d // 8, d % 8, pl.ds(vb * L, L)]
                        plsc.store_scatter(
                            tout_v.at[b],
                            [vb * 2 + rowpat, colpat + d],
                            vec,
                        )
                pltpu.async_copy(tout_v.at[b], _store_slice(p, b), sss[b])

                @pl.when(p + 2 < hi)
                def _():
                    _issue_load(p + 2, b)

        return carry

    lax.fori_loop(0, (PPW + 1) // 2, _body, 0)

    for b in range(2):
        pb = jnp.where(lax.rem(hi - 1 - lo, 2) == b, hi - 1, hi - 2)

        @pl.when(pb >= lo)
        def _():
            pltpu.make_async_copy(
                tout_v.at[b], _store_slice(pb, b), sss[b]
            ).wait()

    @pl.when(wid < F)
    def _():
        f = wid
        for t in range(2):
            pltpu.async_copy(
                nat_hbm.at[
                    pl.ds(pl.multiple_of(f * D + t * 8, 8), 8),
                    pl.ds(VT_LAST * 128, 32),
                ],
                uin_v.at[0, t], sg0,
            )
        for t in range(2):
            pltpu.make_async_copy(
                nat_hbm.at[
                    pl.ds(pl.multiple_of(f * D + t * 8, 8), 8),
                    pl.ds(VT_LAST * 128, 32),
                ],
                uin_v.at[0, t], sg0,
            ).wait()
        for d in range(D):
            for vb in range(2):
                vec = uin_v[0, d // 8, d % 8, pl.ds(vb * L, L)]
                plsc.store_scatter(
                    uout_v.at[0],
                    [vb * 2 + rowpat, colpat + d],
                    vec,
                )
        pltpu.sync_copy(
            uout_v.at[0], out_hbm.at[f, pl.ds(VT_LAST * 16, 4)]
        )


_BM = 2048


def _mlp_body(x_ref, wide_ref, w1_ref, b1_ref, w2_ref, b2_ref, wf_ref, bf_ref,
              o_ref):
    x = x_ref[...]
    h = jnp.maximum(jnp.dot(x, w1_ref[...]) + b1_ref[...], 0.0)
    h = jnp.maximum(jnp.dot(h, w2_ref[...]) + b2_ref[...], 0.0)
    d = jnp.dot(h, wf_ref[...]) + bf_ref[...]
    wsum = jnp.sum(wide_ref[...], axis=1, keepdims=True)
    o_ref[...] = jax.nn.sigmoid(0.5 * wsum + 0.5 * d)


_mlp = pl.pallas_call(
    _mlp_body,
    grid=(B // _BM,),
    in_specs=[
        pl.BlockSpec((_BM, FD), lambda i: (i, 0)),
        pl.BlockSpec((_BM, F), lambda i: (i, 0)),
        pl.BlockSpec((FD, H), lambda i: (0, 0)),
        pl.BlockSpec((1, H), lambda i: (0, 0)),
        pl.BlockSpec((H, H), lambda i: (0, 0)),
        pl.BlockSpec((1, H), lambda i: (0, 0)),
        pl.BlockSpec((H, 1), lambda i: (0, 0)),
        pl.BlockSpec((1, 1), lambda i: (0, 0)),
    ],
    out_specs=pl.BlockSpec((_BM, 1), lambda i: (i, 0)),
    out_shape=jax.ShapeDtypeStruct((B, 1), jnp.float32),
)


def kernel(inputs, embed_tables, W1, b1, W2, b2, Wf, bf, wide_w):
    idx = inputs.astype(jnp.int32).reshape(NW * NCH, CHUNK)
    nat2d = jnp.swapaxes(embed_tables, 1, 2).reshape(F * D, V)
    emb_rows = _sc_transpose(nat2d).reshape(F * V * D // 128, 128)
    wide_rows = wide_w.reshape(F * V // D, D)
    embed_out = _sc_gather(idx, emb_rows)
    wide_out = _sc_wide(idx, wide_rows)
    x = embed_out.reshape(B, FD)
    return _mlp(
        x,
        wide_out.reshape(B, F),
        W1,
        b1.reshape(1, H),
        W2,
        b2.reshape(1, H),
        Wf,
        bf.reshape(1, 1),
    )
